# Initial kernel scaffold; baseline (speedup 1.0000x reference)
import jax
import jax.numpy as jnp
from jax.experimental import pallas as pl


def kernel(x, edge_index, edge_attr, batch, u_soap, u_dimer, Wn1, bn1, Wr1, br1, Wn2, bn2, Wr2, br2, Wn3, bn3, Wr3, br3, Wg1, bg1, Wg2, bg2, Wg3, bg3, Wl1, bl1, Wl2, bl2, Wl3, bl3, Wl, bl):
    raise NotImplementedError("write your pallas kernel here")


# speedup vs baseline: 1.2575x; 1.2575x over previous
import functools

import jax
import jax.numpy as jnp
from jax import lax
from jax.experimental import pallas as pl
from jax.experimental.pallas import tpu as pltpu
from jax.experimental.pallas import tpu_sc as plsc

N = 50000
E = 800000
G = 64
DIN = 35
ED = 43

NPAD = 50176
EPAD = 802816
NBLK = 512
NG = NPAD // NBLK
EBLK = 1024
EG = EPAD // EBLK

C = 128
TILES = 16
EPT = EPAD // TILES
NCH = EPT // C
RPT = NPAD // TILES
ZR = 392
HF = 32


def _q_kernel(ea, w1, w2, w3, q1l, q1h, q2l, q2h, q3l, q3h):
    a = ea[...]
    q1 = jnp.dot(a, w1[...], preferred_element_type=jnp.float32)
    q2 = jnp.dot(a, w2[...], preferred_element_type=jnp.float32)
    q3 = jnp.dot(a, w3[...], preferred_element_type=jnp.float32)
    q1l[...] = q1[:, :HF]
    q1h[...] = q1[:, HF:]
    q2l[...] = q2[:, :HF]
    q2h[...] = q2[:, HF:]
    q3l[...] = q3[:, :HF]
    q3h[...] = q3[:, HF:]


def _compute_q(ea_p, w1, w2, w3):
    out = jax.ShapeDtypeStruct((EPAD, HF), jnp.float32)
    return pl.pallas_call(
        _q_kernel,
        grid=(EG,),
        in_specs=[
            pl.BlockSpec((EBLK, ED), lambda i: (i, 0)),
            pl.BlockSpec((ED, 64), lambda i: (0, 0)),
            pl.BlockSpec((ED, 64), lambda i: (0, 0)),
            pl.BlockSpec((ED, 64), lambda i: (0, 0)),
        ],
        out_specs=[pl.BlockSpec((EBLK, HF), lambda i: (i, 0))] * 6,
        out_shape=[out] * 6,
    )(ea_p, w1, w2, w3)


def _prep1_kernel(x, wnx, bn, wr, br, plo, phi, r):
    xb = x[...]
    p = jnp.dot(xb, wnx[...], preferred_element_type=jnp.float32) + bn[...]
    plo[...] = p[:, :HF]
    phi[...] = p[:, HF:]
    r[...] = jnp.tanh(jnp.dot(xb, wr[...], preferred_element_type=jnp.float32)
                      + br[...])


def _prep1(x_p, wnx, bn, wr, br):
    outs = [jax.ShapeDtypeStruct((NPAD, HF), jnp.float32),
            jax.ShapeDtypeStruct((NPAD, HF), jnp.float32),
            jax.ShapeDtypeStruct((NPAD, 64), jnp.float32)]
    return pl.pallas_call(
        _prep1_kernel,
        grid=(NG,),
        in_specs=[
            pl.BlockSpec((NBLK, DIN), lambda i: (i, 0)),
            pl.BlockSpec((DIN, 64), lambda i: (0, 0)),
            pl.BlockSpec((1, 64), lambda i: (0, 0)),
            pl.BlockSpec((DIN, 64), lambda i: (0, 0)),
            pl.BlockSpec((1, 64), lambda i: (0, 0)),
        ],
        out_specs=[pl.BlockSpec((NBLK, HF), lambda i: (i, 0)),
                   pl.BlockSpec((NBLK, HF), lambda i: (i, 0)),
                   pl.BlockSpec((NBLK, 64), lambda i: (i, 0))],
        out_shape=outs,
    )(x_p, wnx, bn, wr, br)


def _prep23_kernel(rp, al, ah, wnx, bn, wr, br, plo, phi, r):
    h = rp[...] + jnp.concatenate([al[...], ah[...]], axis=1)
    p = jnp.dot(h, wnx[...], preferred_element_type=jnp.float32) + bn[...]
    plo[...] = p[:, :HF]
    phi[...] = p[:, HF:]
    r[...] = jnp.tanh(jnp.dot(h, wr[...], preferred_element_type=jnp.float32)
                      + br[...])


def _prep23(r_prev, acc_lo, acc_hi, wnx, bn, wr, br):
    outs = [jax.ShapeDtypeStruct((NPAD, HF), jnp.float32),
            jax.ShapeDtypeStruct((NPAD, HF), jnp.float32),
            jax.ShapeDtypeStruct((NPAD, 64), jnp.float32)]
    return pl.pallas_call(
        _prep23_kernel,
        grid=(NG,),
        in_specs=[
            pl.BlockSpec((NBLK, 64), lambda i: (i, 0)),
            pl.BlockSpec((NBLK, HF), lambda i: (i, 0)),
            pl.BlockSpec((NBLK, HF), lambda i: (i, 0)),
            pl.BlockSpec((64, 64), lambda i: (0, 0)),
            pl.BlockSpec((1, 64), lambda i: (0, 0)),
            pl.BlockSpec((64, 64), lambda i: (0, 0)),
            pl.BlockSpec((1, 64), lambda i: (0, 0)),
        ],
        out_specs=[pl.BlockSpec((NBLK, HF), lambda i: (i, 0)),
                   pl.BlockSpec((NBLK, HF), lambda i: (i, 0)),
                   pl.BlockSpec((NBLK, 64), lambda i: (i, 0))],
        out_shape=outs,
    )(r_prev, acc_lo, acc_hi, wnx, bn, wr, br)


def _sc_conv_body(src_ref, plo_ref, phi_ref, qlo_ref, qhi_ref,
                  outlo_ref, outhi_ref, idx_v, prow_v, val_v, zb_v,
                  acc_sh, sem):
    c = lax.axis_index("c")
    s = lax.axis_index("s")
    row0 = s * RPT

    zero16 = jnp.zeros((16,), jnp.float32)

    def zfill(i, carry):
        zb_v[i, pl.ds(0, 16)] = zero16
        zb_v[i, pl.ds(16, 16)] = zero16
        return carry

    lax.fori_loop(0, ZR, zfill, 0)

    def zcopy(k, carry):
        pltpu.sync_copy(zb_v, acc_sh.at[pl.ds(row0 + k * ZR, ZR), :])
        return carry

    lax.fori_loop(0, RPT // ZR, zcopy, 0)
    plsc.subcore_barrier()

    def body(p_ref, q_ref, out_ref):
        ebase = s * EPT

        def chunk(j, carry):
            base = ebase + j * C
            pltpu.sync_copy(src_ref.at[pl.ds(base, C)], idx_v)
            pltpu.async_copy(p_ref.at[idx_v], prow_v, sem).wait()
            pltpu.sync_copy(q_ref.at[pl.ds(base, C), :], val_v)

            def cmp(i, cc):
                for f in (0, 16):
                    zx = prow_v[i, pl.ds(f, 16)] + val_v[i, pl.ds(f, 16)]
                    val_v[i, pl.ds(f, 16)] = (
                        1.0 - 2.0 / (jnp.exp(2.0 * zx) + 1.0))
                return cc

            lax.fori_loop(0, C, cmp, 0)
            pltpu.sync_copy(val_v, acc_sh.at[idx_v], add=True)
            return carry

        lax.fori_loop(0, NCH, chunk, 0)
        plsc.subcore_barrier()

        def rback(k, carry):
            r = row0 + k * ZR
            pltpu.sync_copy(acc_sh.at[pl.ds(r, ZR), :], zb_v)
            pltpu.sync_copy(zb_v, out_ref.at[pl.ds(r, ZR), :])
            return carry

        lax.fori_loop(0, RPT // ZR, rback, 0)

    @pl.when(c == 0)
    def _():
        body(plo_ref, qlo_ref, outlo_ref)

    @pl.when(c == 1)
    def _():
        body(phi_ref, qhi_ref, outhi_ref)


def _sc_conv(src_p, p_lo, p_hi, q_lo, q_hi):
    mesh = plsc.VectorSubcoreMesh(core_axis_name="c", subcore_axis_name="s")
    fn = pl.kernel(
        _sc_conv_body,
        out_type=[jax.ShapeDtypeStruct((NPAD, HF), jnp.float32),
                  jax.ShapeDtypeStruct((NPAD, HF), jnp.float32)],
        mesh=mesh,
        scratch_types=[
            pltpu.VMEM((C,), jnp.int32),
            pltpu.VMEM((C, HF), jnp.float32),
            pltpu.VMEM((C, HF), jnp.float32),
            pltpu.VMEM((ZR, HF), jnp.float32),
            pltpu.VMEM_SHARED((NPAD, HF), jnp.float32),
            pltpu.SemaphoreType.DMA,
        ],
        compiler_params=pltpu.CompilerParams(use_tc_tiling_on_sc=False),
    )
    return fn(src_p, p_lo, p_hi, q_lo, q_hi)


def _s1_kernel(r3, al, ah, bt, wg1, bg1, wg2, bg2, wg3, bg3,
               h3, raw, mx_out, mx_sc):
    i = pl.program_id(0)
    h = r3[...] + jnp.concatenate([al[...], ah[...]], axis=1)
    h3[...] = h
    g = jnp.maximum(jnp.dot(h, wg1[...], preferred_element_type=jnp.float32)
                    + bg1[...], 0.0)
    g = jnp.maximum(jnp.dot(g, wg2[...], preferred_element_type=jnp.float32)
                    + bg2[...], 0.0)
    rw = jnp.dot(g, wg3[...], preferred_element_type=jnp.float32) + bg3[...]
    raw[...] = rw
    rows = i * NBLK + lax.broadcasted_iota(jnp.int32, (NBLK, 1), 0)
    valid = rows < N
    onehot = (bt[...] == lax.broadcasted_iota(jnp.int32, (NBLK, G), 1))
    masked = jnp.where(onehot & valid, rw, -1e30)
    bmax = jnp.max(masked, axis=0, keepdims=True)

    @pl.when(i == 0)
    def _():
        mx_sc[...] = jnp.full((1, G), -1e30, jnp.float32)

    mx_sc[...] = jnp.maximum(mx_sc[...], bmax)

    @pl.when(i == NG - 1)
    def _():
        mx_out[...] = mx_sc[...]


def _stage1(r3, acc_lo, acc_hi, batch_p, wg1, bg1, wg2, bg2, wg3, bg3):
    outs = [jax.ShapeDtypeStruct((NPAD, 64), jnp.float32),
            jax.ShapeDtypeStruct((NPAD, 1), jnp.float32),
            jax.ShapeDtypeStruct((1, G), jnp.float32)]
    return pl.pallas_call(
        _s1_kernel,
        grid=(NG,),
        in_specs=[
            pl.BlockSpec((NBLK, 64), lambda i: (i, 0)),
            pl.BlockSpec((NBLK, HF), lambda i: (i, 0)),
            pl.BlockSpec((NBLK, HF), lambda i: (i, 0)),
            pl.BlockSpec((NBLK, 1), lambda i: (i, 0)),
            pl.BlockSpec((64, 32), lambda i: (0, 0)),
            pl.BlockSpec((1, 32), lambda i: (0, 0)),
            pl.BlockSpec((32, 16), lambda i: (0, 0)),
            pl.BlockSpec((1, 16), lambda i: (0, 0)),
            pl.BlockSpec((16, 1), lambda i: (0, 0)),
            pl.BlockSpec((1, 1), lambda i: (0, 0)),
        ],
        out_specs=[pl.BlockSpec((NBLK, 64), lambda i: (i, 0)),
                   pl.BlockSpec((NBLK, 1), lambda i: (i, 0)),
                   pl.BlockSpec((1, G), lambda i: (0, 0))],
        out_shape=outs,
        scratch_shapes=[pltpu.VMEM((1, G), jnp.float32)],
    )(r3, acc_lo, acc_hi, batch_p, wg1, bg1, wg2, bg2, wg3, bg3)


def _s2_kernel(raw, h3, bt, mx, e_out, den_out, u_out, den_sc, u_sc):
    i = pl.program_id(0)
    rows = i * NBLK + lax.broadcasted_iota(jnp.int32, (NBLK, 1), 0)
    valid = rows < N
    onehot = (bt[...] == lax.broadcasted_iota(jnp.int32, (NBLK, G), 1))
    oh = onehot.astype(jnp.float32)
    mrow = jnp.dot(oh, mx[...].reshape(G, 1),
                   preferred_element_type=jnp.float32)
    e = jnp.where(valid, jnp.exp(raw[...] - mrow), 0.0)
    e_out[...] = e
    eh = h3[...] * e

    @pl.when(i == 0)
    def _():
        den_sc[...] = jnp.zeros((1, G), jnp.float32)
        u_sc[...] = jnp.zeros((G, 64), jnp.float32)

    den_sc[...] += lax.dot_general(
        oh, e, (((0,), (0,)), ((), ())),
        preferred_element_type=jnp.float32).reshape(1, G)
    u_sc[...] += lax.dot_general(
        oh, eh, (((0,), (0,)), ((), ())),
        preferred_element_type=jnp.float32)

    @pl.when(i == NG - 1)
    def _():
        den_out[...] = den_sc[...]
        u_out[...] = u_sc[...]


def _stage2(raw, h3, batch_p, mx):
    outs = [jax.ShapeDtypeStruct((NPAD, 1), jnp.float32),
            jax.ShapeDtypeStruct((1, G), jnp.float32),
            jax.ShapeDtypeStruct((G, 64), jnp.float32)]
    return pl.pallas_call(
        _s2_kernel,
        grid=(NG,),
        in_specs=[
            pl.BlockSpec((NBLK, 1), lambda i: (i, 0)),
            pl.BlockSpec((NBLK, 64), lambda i: (i, 0)),
            pl.BlockSpec((NBLK, 1), lambda i: (i, 0)),
            pl.BlockSpec((1, G), lambda i: (0, 0)),
        ],
        out_specs=[pl.BlockSpec((NBLK, 1), lambda i: (i, 0)),
                   pl.BlockSpec((1, G), lambda i: (0, 0)),
                   pl.BlockSpec((G, 64), lambda i: (0, 0))],
        out_shape=outs,
        scratch_shapes=[pltpu.VMEM((1, G), jnp.float32),
                        pltpu.VMEM((G, 64), jnp.float32)],
    )(raw, h3, batch_p, mx)


def _s3_kernel(u, den, us, ud, wl1, bl1, wl2, bl2, wl3, bl3, wl, blb, out):
    recip = 1.0 / (den[...].reshape(G, 1) + 1e-16)
    emb = u[...] * recip
    emb2 = jnp.concatenate([emb, us[...], ud[...]], axis=1)
    o = jnp.maximum(jnp.dot(emb2, wl1[...], preferred_element_type=jnp.float32)
                    + bl1[...], 0.0)
    o = jnp.maximum(jnp.dot(o, wl2[...], preferred_element_type=jnp.float32)
                    + bl2[...], 0.0)
    o = jnp.maximum(jnp.dot(o, wl3[...], preferred_element_type=jnp.float32)
                    + bl3[...], 0.0)
    o4 = jnp.dot(o, wl[...], preferred_element_type=jnp.float32) + blb[...]
    out[...] = jnp.concatenate([o4, jnp.sum(o4, axis=1, keepdims=True)],
                               axis=1)


def _stage3(u, den, us, ud, wl1, bl1, wl2, bl2, wl3, bl3, wl, blb):
    return pl.pallas_call(
        _s3_kernel,
        out_shape=jax.ShapeDtypeStruct((G, 5), jnp.float32),
    )(u, den, us, ud, wl1, bl1, wl2, bl2, wl3, bl3, wl, blb)


def _s4_kernel(e, bt, den, gate):
    onehot = (bt[...] == lax.broadcasted_iota(jnp.int32, (NBLK, G), 1))
    oh = onehot.astype(jnp.float32)
    recip = 1.0 / (den[...].reshape(G, 1) + 1e-16)
    d = jnp.dot(oh, recip, preferred_element_type=jnp.float32)
    gate[...] = e[...] * d


def _stage4(e, batch_p, den):
    return pl.pallas_call(
        _s4_kernel,
        grid=(NG,),
        in_specs=[
            pl.BlockSpec((NBLK, 1), lambda i: (i, 0)),
            pl.BlockSpec((NBLK, 1), lambda i: (i, 0)),
            pl.BlockSpec((1, G), lambda i: (0, 0)),
        ],
        out_specs=pl.BlockSpec((NBLK, 1), lambda i: (i, 0)),
        out_shape=jax.ShapeDtypeStruct((NPAD, 1), jnp.float32),
    )(e, batch_p, den)


def kernel(x, edge_index, edge_attr, batch, u_soap, u_dimer,
           Wn1, bn1, Wr1, br1, Wn2, bn2, Wr2, br2, Wn3, bn3, Wr3, br3,
           Wg1, bg1, Wg2, bg2, Wg3, bg3,
           Wl1, bl1, Wl2, bl2, Wl3, bl3, Wl, bl):
    src_p = jnp.pad(edge_index[0].astype(jnp.int32), (0, EPAD - E),
                    constant_values=N)
    ea_p = jnp.pad(edge_attr, ((0, EPAD - E), (0, 0)))
    x_p = jnp.pad(x, ((0, NPAD - N), (0, 0)))
    batch_p = jnp.pad(batch.astype(jnp.int32), (0, NPAD - N),
                      constant_values=G - 1).reshape(NPAD, 1)

    b = lambda v: v.reshape(1, -1)

    q1l, q1h, q2l, q2h, q3l, q3h = _compute_q(
        ea_p, Wn1[DIN:], Wn2[64:], Wn3[64:])

    p1l, p1h, r1 = _prep1(x_p, Wn1[:DIN], b(bn1), Wr1, b(br1))
    a1l, a1h = _sc_conv(src_p, p1l, p1h, q1l, q1h)

    p2l, p2h, r2 = _prep23(r1, a1l, a1h, Wn2[:64], b(bn2), Wr2, b(br2))
    a2l, a2h = _sc_conv(src_p, p2l, p2h, q2l, q2h)

    p3l, p3h, r3 = _prep23(r2, a2l, a2h, Wn3[:64], b(bn3), Wr3, b(br3))
    a3l, a3h = _sc_conv(src_p, p3l, p3h, q3l, q3h)

    h3, raw, mx = _stage1(r3, a3l, a3h, batch_p,
                          Wg1, b(bg1), Wg2, b(bg2), Wg3, b(bg3))
    e, den, u = _stage2(raw, h3, batch_p, mx)
    out = _stage3(u, den, u_soap, u_dimer,
                  Wl1, b(bl1), Wl2, b(bl2), Wl3, b(bl3), Wl, b(bl))
    gate = _stage4(e, batch_p, den)[:N]
    return (out, gate)


# speedup vs baseline: 2.4598x; 1.9561x over previous
import functools

import jax
import jax.numpy as jnp
from jax import lax
from jax.experimental import pallas as pl
from jax.experimental.pallas import tpu as pltpu
from jax.experimental.pallas import tpu_sc as plsc

N = 50000
E = 800000
G = 64
DIN = 35
ED = 43

NPAD = 50176
EPAD = 802816
NBLK = 512
NG = NPAD // NBLK
EBLK = 1024
EG = EPAD // EBLK

C = 112
TILES = 16
EPT = EPAD // TILES
NCH = EPT // C
RPT = NPAD // TILES
ZR = 112
HF = 32


def _q_kernel(ea, w1, w2, w3, q1l, q1h, q2l, q2h, q3l, q3h):
    a = ea[...]
    q1 = jnp.dot(a, w1[...], preferred_element_type=jnp.float32)
    q2 = jnp.dot(a, w2[...], preferred_element_type=jnp.float32)
    q3 = jnp.dot(a, w3[...], preferred_element_type=jnp.float32)
    q1l[...] = q1[:, :HF]
    q1h[...] = q1[:, HF:]
    q2l[...] = q2[:, :HF]
    q2h[...] = q2[:, HF:]
    q3l[...] = q3[:, :HF]
    q3h[...] = q3[:, HF:]


def _compute_q(ea_p, w1, w2, w3):
    out = jax.ShapeDtypeStruct((EPAD, HF), jnp.float32)
    return pl.pallas_call(
        _q_kernel,
        grid=(EG,),
        in_specs=[
            pl.BlockSpec((EBLK, ED), lambda i: (i, 0)),
            pl.BlockSpec((ED, 64), lambda i: (0, 0)),
            pl.BlockSpec((ED, 64), lambda i: (0, 0)),
            pl.BlockSpec((ED, 64), lambda i: (0, 0)),
        ],
        out_specs=[pl.BlockSpec((EBLK, HF), lambda i: (i, 0))] * 6,
        out_shape=[out] * 6,
    )(ea_p, w1, w2, w3)


def _prep1_kernel(x, wnx, bn, wr, br, plo, phi, r):
    xb = x[...]
    p = jnp.dot(xb, wnx[...], preferred_element_type=jnp.float32) + bn[...]
    plo[...] = p[:, :HF]
    phi[...] = p[:, HF:]
    r[...] = jnp.tanh(jnp.dot(xb, wr[...], preferred_element_type=jnp.float32)
                      + br[...])


def _prep1(x_p, wnx, bn, wr, br):
    outs = [jax.ShapeDtypeStruct((NPAD, HF), jnp.float32),
            jax.ShapeDtypeStruct((NPAD, HF), jnp.float32),
            jax.ShapeDtypeStruct((NPAD, 64), jnp.float32)]
    return pl.pallas_call(
        _prep1_kernel,
        grid=(NG,),
        in_specs=[
            pl.BlockSpec((NBLK, DIN), lambda i: (i, 0)),
            pl.BlockSpec((DIN, 64), lambda i: (0, 0)),
            pl.BlockSpec((1, 64), lambda i: (0, 0)),
            pl.BlockSpec((DIN, 64), lambda i: (0, 0)),
            pl.BlockSpec((1, 64), lambda i: (0, 0)),
        ],
        out_specs=[pl.BlockSpec((NBLK, HF), lambda i: (i, 0)),
                   pl.BlockSpec((NBLK, HF), lambda i: (i, 0)),
                   pl.BlockSpec((NBLK, 64), lambda i: (i, 0))],
        out_shape=outs,
    )(x_p, wnx, bn, wr, br)


def _prep23_kernel(rp, al, ah, wnx, bn, wr, br, plo, phi, r):
    h = rp[...] + jnp.concatenate([al[...], ah[...]], axis=1)
    p = jnp.dot(h, wnx[...], preferred_element_type=jnp.float32) + bn[...]
    plo[...] = p[:, :HF]
    phi[...] = p[:, HF:]
    r[...] = jnp.tanh(jnp.dot(h, wr[...], preferred_element_type=jnp.float32)
                      + br[...])


def _prep23(r_prev, acc_lo, acc_hi, wnx, bn, wr, br):
    outs = [jax.ShapeDtypeStruct((NPAD, HF), jnp.float32),
            jax.ShapeDtypeStruct((NPAD, HF), jnp.float32),
            jax.ShapeDtypeStruct((NPAD, 64), jnp.float32)]
    return pl.pallas_call(
        _prep23_kernel,
        grid=(NG,),
        in_specs=[
            pl.BlockSpec((NBLK, 64), lambda i: (i, 0)),
            pl.BlockSpec((NBLK, HF), lambda i: (i, 0)),
            pl.BlockSpec((NBLK, HF), lambda i: (i, 0)),
            pl.BlockSpec((64, 64), lambda i: (0, 0)),
            pl.BlockSpec((1, 64), lambda i: (0, 0)),
            pl.BlockSpec((64, 64), lambda i: (0, 0)),
            pl.BlockSpec((1, 64), lambda i: (0, 0)),
        ],
        out_specs=[pl.BlockSpec((NBLK, HF), lambda i: (i, 0)),
                   pl.BlockSpec((NBLK, HF), lambda i: (i, 0)),
                   pl.BlockSpec((NBLK, 64), lambda i: (i, 0))],
        out_shape=outs,
    )(r_prev, acc_lo, acc_hi, wnx, bn, wr, br)


def _sc_conv_body(src_ref, plo_ref, phi_ref, qlo_ref, qhi_ref,
                  outlo_ref, outhi_ref,
                  idx0, idx1, idx2, idx3,
                  prow0, prow1, qv0, qv1, val0, val1, zb_v, acc_sh,
                  sg0, sg1, sq0, sq1, ss0, ss1):
    c = lax.axis_index("c")
    s = lax.axis_index("s")
    row0 = s * RPT

    IDX = (idx0, idx1, idx2, idx3)
    PROW = (prow0, prow1)
    QV = (qv0, qv1)
    VAL = (val0, val1)
    SG = (sg0, sg1)
    SQ = (sq0, sq1)
    SS = (ss0, ss1)

    zero16 = jnp.zeros((16,), jnp.float32)

    def zfill(i, carry):
        zb_v[i, pl.ds(0, 16)] = zero16
        zb_v[i, pl.ds(16, 16)] = zero16
        return carry

    lax.fori_loop(0, ZR, zfill, 0)

    def zcopy(k, carry):
        pltpu.sync_copy(zb_v, acc_sh.at[pl.ds(row0 + k * ZR, ZR), :])
        return carry

    lax.fori_loop(0, RPT // ZR, zcopy, 0)
    plsc.subcore_barrier()

    def body(p_ref, q_ref, out_ref):
        ebase = s * EPT

        def fetch(j, i4, b):
            base = ebase + j * C
            pltpu.sync_copy(src_ref.at[pl.ds(base, C)], IDX[i4])
            pltpu.async_copy(p_ref.at[IDX[i4]], PROW[b], SG[b])
            pltpu.async_copy(q_ref.at[pl.ds(base, C), :], QV[b], SQ[b])

        fetch(0, 0, 0)
        fetch(1, 1, 1)

        def sweep(jo, carry):
            for u in range(4):
                b = u % 2
                j = jo * 4 + u
                pltpu.make_async_copy(
                    p_ref.at[IDX[u]], PROW[b], SG[b]).wait()
                pltpu.make_async_copy(
                    q_ref.at[pl.ds(0, C), :], QV[b], SQ[b]).wait()

                def swait():
                    pltpu.make_async_copy(
                        VAL[b], acc_sh.at[IDX[u]], SS[b]).wait()

                if u >= 2:
                    swait()
                else:
                    @pl.when(jo > 0)
                    def _():
                        swait()

                prow_v, q_v, v_v = PROW[b], QV[b], VAL[b]

                def cmp(i, cc):
                    for f in (0, 16):
                        zx = prow_v[i, pl.ds(f, 16)] + q_v[i, pl.ds(f, 16)]
                        v_v[i, pl.ds(f, 16)] = (
                            1.0 - 2.0 / (jnp.exp(2.0 * zx) + 1.0))
                    return cc

                lax.fori_loop(0, C, cmp, 0)
                pltpu.async_copy(VAL[b], acc_sh.at[IDX[u]], SS[b], add=True)

                if u < 2:
                    fetch(j + 2, u + 2, b)
                else:
                    @pl.when(jo < NCH // 4 - 1)
                    def _():
                        fetch(j + 2, u - 2, b)
            return carry

        lax.fori_loop(0, NCH // 4, sweep, 0)
        for b in range(2):
            pltpu.make_async_copy(VAL[b], acc_sh.at[IDX[b]], SS[b]).wait()
        plsc.subcore_barrier()

        def rback(k, carry):
            r = row0 + k * ZR
            pltpu.sync_copy(acc_sh.at[pl.ds(r, ZR), :], zb_v)
            pltpu.sync_copy(zb_v, out_ref.at[pl.ds(r, ZR), :])
            return carry

        lax.fori_loop(0, RPT // ZR, rback, 0)

    @pl.when(c == 0)
    def _():
        body(plo_ref, qlo_ref, outlo_ref)

    @pl.when(c == 1)
    def _():
        body(phi_ref, qhi_ref, outhi_ref)


def _sc_conv(src_p, p_lo, p_hi, q_lo, q_hi):
    mesh = plsc.VectorSubcoreMesh(core_axis_name="c", subcore_axis_name="s")
    fn = pl.kernel(
        _sc_conv_body,
        out_type=[jax.ShapeDtypeStruct((NPAD, HF), jnp.float32),
                  jax.ShapeDtypeStruct((NPAD, HF), jnp.float32)],
        mesh=mesh,
        scratch_types=(
            [pltpu.VMEM((C,), jnp.int32)] * 4
            + [pltpu.VMEM((C, HF), jnp.float32)] * 6
            + [pltpu.VMEM((ZR, HF), jnp.float32),
               pltpu.VMEM_SHARED((NPAD, HF), jnp.float32)]
            + [pltpu.SemaphoreType.DMA] * 6
        ),
        compiler_params=pltpu.CompilerParams(use_tc_tiling_on_sc=False),
    )
    return fn(src_p, p_lo, p_hi, q_lo, q_hi)


def _s1_kernel(r3, al, ah, bt, wg1, bg1, wg2, bg2, wg3, bg3,
               h3, raw, mx_out, mx_sc):
    i = pl.program_id(0)
    h = r3[...] + jnp.concatenate([al[...], ah[...]], axis=1)
    h3[...] = h
    g = jnp.maximum(jnp.dot(h, wg1[...], preferred_element_type=jnp.float32)
                    + bg1[...], 0.0)
    g = jnp.maximum(jnp.dot(g, wg2[...], preferred_element_type=jnp.float32)
                    + bg2[...], 0.0)
    rw = jnp.dot(g, wg3[...], preferred_element_type=jnp.float32) + bg3[...]
    raw[...] = rw
    rows = i * NBLK + lax.broadcasted_iota(jnp.int32, (NBLK, 1), 0)
    valid = rows < N
    onehot = (bt[...] == lax.broadcasted_iota(jnp.int32, (NBLK, G), 1))
    masked = jnp.where(onehot & valid, rw, -1e30)
    bmax = jnp.max(masked, axis=0, keepdims=True)

    @pl.when(i == 0)
    def _():
        mx_sc[...] = jnp.full((1, G), -1e30, jnp.float32)

    mx_sc[...] = jnp.maximum(mx_sc[...], bmax)

    @pl.when(i == NG - 1)
    def _():
        mx_out[...] = mx_sc[...]


def _stage1(r3, acc_lo, acc_hi, batch_p, wg1, bg1, wg2, bg2, wg3, bg3):
    outs = [jax.ShapeDtypeStruct((NPAD, 64), jnp.float32),
            jax.ShapeDtypeStruct((NPAD, 1), jnp.float32),
            jax.ShapeDtypeStruct((1, G), jnp.float32)]
    return pl.pallas_call(
        _s1_kernel,
        grid=(NG,),
        in_specs=[
            pl.BlockSpec((NBLK, 64), lambda i: (i, 0)),
            pl.BlockSpec((NBLK, HF), lambda i: (i, 0)),
            pl.BlockSpec((NBLK, HF), lambda i: (i, 0)),
            pl.BlockSpec((NBLK, 1), lambda i: (i, 0)),
            pl.BlockSpec((64, 32), lambda i: (0, 0)),
            pl.BlockSpec((1, 32), lambda i: (0, 0)),
            pl.BlockSpec((32, 16), lambda i: (0, 0)),
            pl.BlockSpec((1, 16), lambda i: (0, 0)),
            pl.BlockSpec((16, 1), lambda i: (0, 0)),
            pl.BlockSpec((1, 1), lambda i: (0, 0)),
        ],
        out_specs=[pl.BlockSpec((NBLK, 64), lambda i: (i, 0)),
                   pl.BlockSpec((NBLK, 1), lambda i: (i, 0)),
                   pl.BlockSpec((1, G), lambda i: (0, 0))],
        out_shape=outs,
        scratch_shapes=[pltpu.VMEM((1, G), jnp.float32)],
    )(r3, acc_lo, acc_hi, batch_p, wg1, bg1, wg2, bg2, wg3, bg3)


def _s2_kernel(raw, h3, bt, mx, e_out, den_out, u_out, den_sc, u_sc):
    i = pl.program_id(0)
    rows = i * NBLK + lax.broadcasted_iota(jnp.int32, (NBLK, 1), 0)
    valid = rows < N
    onehot = (bt[...] == lax.broadcasted_iota(jnp.int32, (NBLK, G), 1))
    oh = onehot.astype(jnp.float32)
    mrow = jnp.dot(oh, mx[...].reshape(G, 1),
                   preferred_element_type=jnp.float32)
    e = jnp.where(valid, jnp.exp(raw[...] - mrow), 0.0)
    e_out[...] = e
    eh = h3[...] * e

    @pl.when(i == 0)
    def _():
        den_sc[...] = jnp.zeros((1, G), jnp.float32)
        u_sc[...] = jnp.zeros((G, 64), jnp.float32)

    den_sc[...] += lax.dot_general(
        oh, e, (((0,), (0,)), ((), ())),
        preferred_element_type=jnp.float32).reshape(1, G)
    u_sc[...] += lax.dot_general(
        oh, eh, (((0,), (0,)), ((), ())),
        preferred_element_type=jnp.float32)

    @pl.when(i == NG - 1)
    def _():
        den_out[...] = den_sc[...]
        u_out[...] = u_sc[...]


def _stage2(raw, h3, batch_p, mx):
    outs = [jax.ShapeDtypeStruct((NPAD, 1), jnp.float32),
            jax.ShapeDtypeStruct((1, G), jnp.float32),
            jax.ShapeDtypeStruct((G, 64), jnp.float32)]
    return pl.pallas_call(
        _s2_kernel,
        grid=(NG,),
        in_specs=[
            pl.BlockSpec((NBLK, 1), lambda i: (i, 0)),
            pl.BlockSpec((NBLK, 64), lambda i: (i, 0)),
            pl.BlockSpec((NBLK, 1), lambda i: (i, 0)),
            pl.BlockSpec((1, G), lambda i: (0, 0)),
        ],
        out_specs=[pl.BlockSpec((NBLK, 1), lambda i: (i, 0)),
                   pl.BlockSpec((1, G), lambda i: (0, 0)),
                   pl.BlockSpec((G, 64), lambda i: (0, 0))],
        out_shape=outs,
        scratch_shapes=[pltpu.VMEM((1, G), jnp.float32),
                        pltpu.VMEM((G, 64), jnp.float32)],
    )(raw, h3, batch_p, mx)


def _s3_kernel(u, den, us, ud, wl1, bl1, wl2, bl2, wl3, bl3, wl, blb, out):
    recip = 1.0 / (den[...].reshape(G, 1) + 1e-16)
    emb = u[...] * recip
    emb2 = jnp.concatenate([emb, us[...], ud[...]], axis=1)
    o = jnp.maximum(jnp.dot(emb2, wl1[...], preferred_element_type=jnp.float32)
                    + bl1[...], 0.0)
    o = jnp.maximum(jnp.dot(o, wl2[...], preferred_element_type=jnp.float32)
                    + bl2[...], 0.0)
    o = jnp.maximum(jnp.dot(o, wl3[...], preferred_element_type=jnp.float32)
                    + bl3[...], 0.0)
    o4 = jnp.dot(o, wl[...], preferred_element_type=jnp.float32) + blb[...]
    out[...] = jnp.concatenate([o4, jnp.sum(o4, axis=1, keepdims=True)],
                               axis=1)


def _stage3(u, den, us, ud, wl1, bl1, wl2, bl2, wl3, bl3, wl, blb):
    return pl.pallas_call(
        _s3_kernel,
        out_shape=jax.ShapeDtypeStruct((G, 5), jnp.float32),
    )(u, den, us, ud, wl1, bl1, wl2, bl2, wl3, bl3, wl, blb)


def _s4_kernel(e, bt, den, gate):
    onehot = (bt[...] == lax.broadcasted_iota(jnp.int32, (NBLK, G), 1))
    oh = onehot.astype(jnp.float32)
    recip = 1.0 / (den[...].reshape(G, 1) + 1e-16)
    d = jnp.dot(oh, recip, preferred_element_type=jnp.float32)
    gate[...] = e[...] * d


def _stage4(e, batch_p, den):
    return pl.pallas_call(
        _s4_kernel,
        grid=(NG,),
        in_specs=[
            pl.BlockSpec((NBLK, 1), lambda i: (i, 0)),
            pl.BlockSpec((NBLK, 1), lambda i: (i, 0)),
            pl.BlockSpec((1, G), lambda i: (0, 0)),
        ],
        out_specs=pl.BlockSpec((NBLK, 1), lambda i: (i, 0)),
        out_shape=jax.ShapeDtypeStruct((NPAD, 1), jnp.float32),
    )(e, batch_p, den)


def kernel(x, edge_index, edge_attr, batch, u_soap, u_dimer,
           Wn1, bn1, Wr1, br1, Wn2, bn2, Wr2, br2, Wn3, bn3, Wr3, br3,
           Wg1, bg1, Wg2, bg2, Wg3, bg3,
           Wl1, bl1, Wl2, bl2, Wl3, bl3, Wl, bl):
    src_p = jnp.pad(edge_index[0].astype(jnp.int32), (0, EPAD - E),
                    constant_values=N)
    ea_p = jnp.pad(edge_attr, ((0, EPAD - E), (0, 0)))
    x_p = jnp.pad(x, ((0, NPAD - N), (0, 0)))
    batch_p = jnp.pad(batch.astype(jnp.int32), (0, NPAD - N),
                      constant_values=G - 1).reshape(NPAD, 1)

    b = lambda v: v.reshape(1, -1)

    q1l, q1h, q2l, q2h, q3l, q3h = _compute_q(
        ea_p, Wn1[DIN:], Wn2[64:], Wn3[64:])

    p1l, p1h, r1 = _prep1(x_p, Wn1[:DIN], b(bn1), Wr1, b(br1))
    a1l, a1h = _sc_conv(src_p, p1l, p1h, q1l, q1h)

    p2l, p2h, r2 = _prep23(r1, a1l, a1h, Wn2[:64], b(bn2), Wr2, b(br2))
    a2l, a2h = _sc_conv(src_p, p2l, p2h, q2l, q2h)

    p3l, p3h, r3 = _prep23(r2, a2l, a2h, Wn3[:64], b(bn3), Wr3, b(br3))
    a3l, a3h = _sc_conv(src_p, p3l, p3h, q3l, q3h)

    h3, raw, mx = _stage1(r3, a3l, a3h, batch_p,
                          Wg1, b(bg1), Wg2, b(bg2), Wg3, b(bg3))
    e, den, u = _stage2(raw, h3, batch_p, mx)
    out = _stage3(u, den, u_soap, u_dimer,
                  Wl1, b(bl1), Wl2, b(bl2), Wl3, b(bl3), Wl, b(bl))
    gate = _stage4(e, batch_p, den)[:N]
    return (out, gate)
